# Initial kernel scaffold; baseline (speedup 1.0000x reference)
import functools

import jax
import jax.numpy as jnp
from jax import lax
from jax.experimental import pallas as pl
from jax.experimental.pallas import tpu as pltpu
from jax.experimental.pallas import tpu_sc as plsc

_N = 50000
_NP = 50176
_E = 800000
_K = 40000
_NS = 16
_NCH = _NP // _NS
_TE = _E // _NS
_NB = _TE // 128
_REM = _TE - _NB * 128
_TED = _E // 2 // _NS
_NBD = _TED // 128
_REMD = _TED - _NBD * 128
_R = 512
_G = _NP // _R
_NEG = -1e30

_sc_mesh = plsc.VectorSubcoreMesh(
    core_axis_name="c", subcore_axis_name="s", num_cores=2, num_subcores=_NS)


def _zero_rows32(ref, n):
    def body(i, _):
        ref[i, pl.ds(0, 16)] = jnp.zeros((16,), jnp.float32)
        ref[i, pl.ds(16, 16)] = jnp.zeros((16,), jnp.float32)
        return 0
    lax.fori_loop(0, n, body, 0)


def _fill1d(ref, n, val, dtype):
    def body(i, _):
        ref[pl.ds(i * 16, 16)] = jnp.full((16,), val, dtype)
        return 0
    lax.fori_loop(0, n // 16, body, 0)


def _deg_body(dst_hbm, degp, dacc, zbuf, onesv, dstv, onesv2, dstv2):
    c = lax.axis_index("c")
    s = lax.axis_index("s")
    _fill1d(zbuf, 1024, 0.0, jnp.float32)
    _fill1d(onesv, 128, 1.0, jnp.float32)
    _fill1d(onesv2, 48, 1.0, jnp.float32)
    zbase = s * _NCH
    for t in range(_NCH // 1024):
        pltpu.sync_copy(zbuf, dacc.at[pl.ds(zbase + t * 1024, 1024)])
    rem = _NCH % 1024
    if rem:
        pltpu.sync_copy(zbuf.at[pl.ds(0, rem)],
                        dacc.at[pl.ds(zbase + (_NCH // 1024) * 1024, rem)])
    plsc.subcore_barrier()
    tile_base = c * (_E // 2) + s * _TED
    def step(j, _):
        pltpu.sync_copy(dst_hbm.at[pl.ds(tile_base + j * 128, 128)], dstv)
        pltpu.sync_copy(onesv, dacc.at[dstv], add=True)
        return 0
    lax.fori_loop(0, _NBD, step, 0)
    _fill1d(dstv2, 48, _N, jnp.int32)
    pltpu.sync_copy(dst_hbm.at[pl.ds(tile_base + _NBD * 128, _REMD)],
                    dstv2.at[pl.ds(0, _REMD)])
    pltpu.sync_copy(onesv2, dacc.at[dstv2], add=True)
    plsc.subcore_barrier()
    pltpu.sync_copy(dacc.at[pl.ds(zbase, _NCH)], degp.at[c, pl.ds(zbase, _NCH)])


_k_deg = pl.kernel(
    _deg_body,
    out_type=jax.ShapeDtypeStruct((2, _NP), jnp.float32),
    mesh=_sc_mesh,
    scratch_types=[
        pltpu.VMEM_SHARED((_NP,), jnp.float32),
        pltpu.VMEM((1024,), jnp.float32),
        pltpu.VMEM((128,), jnp.float32),
        pltpu.VMEM((128,), jnp.int32),
        pltpu.VMEM((48,), jnp.float32),
        pltpu.VMEM((48,), jnp.int32),
    ],
)


def _agg_body(hr, srch, dsth, out,
              acc, zrows, srcv, dstv, gidxv, rows,
              srcv2, dstv2, gidxv2, rows2, gsem):
    c = lax.axis_index("c")
    s = lax.axis_index("s")
    _zero_rows32(zrows, 128)
    zbase = s * _NCH
    for t in range(_NCH // 128):
        pltpu.sync_copy(zrows, acc.at[pl.ds(zbase + t * 128, 128)])
    rem = _NCH % 128
    if rem:
        pltpu.sync_copy(zrows.at[pl.ds(0, rem)],
                        acc.at[pl.ds(zbase + (_NCH // 128) * 128, rem)])
    plsc.subcore_barrier()
    tile_base = s * _TE
    def step(j, _):
        base = tile_base + j * 128
        pltpu.sync_copy(srch.at[pl.ds(base, 128)], srcv)
        pltpu.sync_copy(dsth.at[pl.ds(base, 128)], dstv)
        def gi(kk, _2):
            gidxv[pl.ds(kk * 16, 16)] = srcv[pl.ds(kk * 16, 16)] * 2 + c
            return 0
        lax.fori_loop(0, 8, gi, 0)
        pltpu.async_copy(hr.at[gidxv], rows, gsem).wait()
        pltpu.sync_copy(rows, acc.at[dstv], add=True)
        return 0
    lax.fori_loop(0, _NB, step, 0)
    base = tile_base + _NB * 128
    pltpu.sync_copy(srch.at[pl.ds(base, _REM)], srcv2)
    pltpu.sync_copy(dsth.at[pl.ds(base, _REM)], dstv2)
    def gi2(kk, _2):
        gidxv2[pl.ds(kk * 16, 16)] = srcv2[pl.ds(kk * 16, 16)] * 2 + c
        return 0
    lax.fori_loop(0, _REM // 16, gi2, 0)
    pltpu.async_copy(hr.at[gidxv2], rows2, gsem).wait()
    pltpu.sync_copy(rows2, acc.at[dstv2], add=True)
    plsc.subcore_barrier()
    pltpu.sync_copy(acc.at[pl.ds(zbase, _NCH)], out.at[c, pl.ds(zbase, _NCH)])


_k_agg = pl.kernel(
    _agg_body,
    out_type=jax.ShapeDtypeStruct((2, _NP, 32), jnp.float32),
    mesh=_sc_mesh,
    scratch_types=[
        pltpu.VMEM_SHARED((_NP, 32), jnp.float32),
        pltpu.VMEM((128, 32), jnp.float32),
        pltpu.VMEM((128,), jnp.int32),
        pltpu.VMEM((128,), jnp.int32),
        pltpu.VMEM((128,), jnp.int32),
        pltpu.VMEM((128, 32), jnp.float32),
        pltpu.VMEM((_REM,), jnp.int32),
        pltpu.VMEM((_REM,), jnp.int32),
        pltpu.VMEM((_REM,), jnp.int32),
        pltpu.VMEM((_REM, 32), jnp.float32),
        pltpu.SemaphoreType.DMA,
    ],
)


def _mm1_body(xref, wref, dref, htref, dinvref):
    i = pl.program_id(0)
    dt = jnp.transpose(dref[...])
    deg = dt[:, 0:1] + dt[:, 1:2] + 1.0
    rows = lax.broadcasted_iota(jnp.int32, (_R, 1), 0) + i * _R
    dinv = jnp.where(rows < _N, lax.rsqrt(deg), 0.0)
    dinvref[...] = dinv
    h = jnp.dot(xref[...], wref[...], preferred_element_type=jnp.float32)
    htref[...] = h * dinv


_k_mm1 = pl.pallas_call(
    _mm1_body,
    grid=(_G,),
    in_specs=[
        pl.BlockSpec((_R, 128), lambda i: (i, 0)),
        pl.BlockSpec((128, 64), lambda i: (0, 0)),
        pl.BlockSpec((2, _R), lambda i: (0, i)),
    ],
    out_specs=[
        pl.BlockSpec((_R, 64), lambda i: (i, 0)),
        pl.BlockSpec((_R, 1), lambda i: (i, 0)),
    ],
    out_shape=[
        jax.ShapeDtypeStruct((_NP, 64), jnp.float32),
        jax.ShapeDtypeStruct((_NP, 1), jnp.float32),
    ],
)


def _g2_body(aref, htref, dinvref, bref, gref):
    i = pl.program_id(0)
    ab = aref[...]
    agg = jnp.concatenate([ab[0], ab[1]], axis=1)
    rows = lax.broadcasted_iota(jnp.int32, (_R, 1), 0) + i * _R
    agg = jnp.where(rows < _N, agg, 0.0)
    dinv = dinvref[...]
    h1 = jnp.maximum(dinv * (agg + htref[...]) + bref[...].reshape(1, 64), 0.0)
    gref[...] = h1 * dinv


_k_g2 = pl.pallas_call(
    _g2_body,
    grid=(_G,),
    in_specs=[
        pl.BlockSpec((2, _R, 32), lambda i: (0, i, 0)),
        pl.BlockSpec((_R, 64), lambda i: (i, 0)),
        pl.BlockSpec((_R, 1), lambda i: (i, 0)),
        pl.BlockSpec((64,), lambda i: (0,)),
    ],
    out_specs=pl.BlockSpec((_R, 64), lambda i: (i, 0)),
    out_shape=jax.ShapeDtypeStruct((_NP, 64), jnp.float32),
)


def _h2_body(aref, gref, dinvref, wref, bref, pref, h2ref, scref):
    i = pl.program_id(0)
    ab = aref[...]
    agg = jnp.concatenate([ab[0], ab[1]], axis=1)
    rows = lax.broadcasted_iota(jnp.int32, (_R, 1), 0) + i * _R
    agg = jnp.where(rows < _N, agg, 0.0)
    dinv = dinvref[...]
    pre = dinv * (agg + gref[...])
    h2 = jnp.dot(pre, wref[...], preferred_element_type=jnp.float32)
    h2 = jnp.maximum(h2 + bref[...].reshape(1, 128), 0.0)
    h2 = jnp.where(rows < _N, h2, 0.0)
    h2ref[...] = h2
    pvec = pref[...].reshape(1, 128)
    pnorm = jnp.sqrt(jnp.sum(pvec * pvec))
    pcol = jnp.transpose(pvec)
    sc = jnp.dot(h2, pcol, preferred_element_type=jnp.float32) / pnorm
    scref[...] = jnp.where(rows < _N, sc, _NEG)


_k_h2 = pl.pallas_call(
    _h2_body,
    grid=(_G,),
    in_specs=[
        pl.BlockSpec((2, _R, 32), lambda i: (0, i, 0)),
        pl.BlockSpec((_R, 64), lambda i: (i, 0)),
        pl.BlockSpec((_R, 1), lambda i: (i, 0)),
        pl.BlockSpec((64, 128), lambda i: (0, 0)),
        pl.BlockSpec((128,), lambda i: (0,)),
        pl.BlockSpec((128,), lambda i: (0,)),
    ],
    out_specs=[
        pl.BlockSpec((_R, 128), lambda i: (i, 0)),
        pl.BlockSpec((_R, 1), lambda i: (i, 0)),
    ],
    out_shape=[
        jax.ShapeDtypeStruct((_NP, 128), jnp.float32),
        jax.ShapeDtypeStruct((_NP, 1), jnp.float32),
    ],
)


def _sel_body(sref, wref, mref):
    s = sref[...]
    ub = lax.bitcast_convert_type(s, jnp.uint32)
    key = jnp.where((ub >> 31) != 0, ~ub, ub | jnp.uint32(0x80000000))
    t = jnp.uint32(0)
    for b in range(31, -1, -1):
        cand = t | (jnp.uint32(1) << b)
        cnt = jnp.sum((key >= cand).astype(jnp.int32))
        t = jnp.where(cnt >= _K, cand, t)
    cgt = jnp.sum((key > t).astype(jnp.int32))
    need = _K - cgt
    eq = key == t
    idx = (lax.broadcasted_iota(jnp.int32, (392, 128), 0) * 128
           + lax.broadcasted_iota(jnp.int32, (392, 128), 1))
    J = jnp.int32(0)
    for b in range(16, -1, -1):
        cand = J + jnp.int32(1 << b)
        cnt = jnp.sum((eq & (idx < cand)).astype(jnp.int32))
        J = jnp.where(cnt <= need, cand, J)
    sel = (key > t) | (eq & (idx < J))
    wref[...] = jnp.tanh(s)
    mref[...] = jnp.where(sel, 0.0, _NEG)


_k_sel = pl.pallas_call(
    _sel_body,
    grid=(1,),
    in_specs=[pl.BlockSpec((392, 128), lambda i: (0, 0))],
    out_specs=[
        pl.BlockSpec((392, 128), lambda i: (0, 0)),
        pl.BlockSpec((392, 128), lambda i: (0, 0)),
    ],
    out_shape=[
        jax.ShapeDtypeStruct((392, 128), jnp.float32),
        jax.ShapeDtypeStruct((392, 128), jnp.float32),
    ],
)


def _pool_body(h2ref, wref, mref, fwref, fbref, oref, acc):
    i = pl.program_id(0)

    @pl.when(i == 0)
    def _init():
        acc[...] = jnp.full((1, 128), _NEG, jnp.float32)

    wcol = jnp.transpose(wref[...])
    mcol = jnp.transpose(mref[...])
    contrib = h2ref[...] * wcol + mcol
    acc[...] = jnp.maximum(acc[...], jnp.max(contrib, axis=0, keepdims=True))

    @pl.when(i == 392 - 1)
    def _fin():
        pooled = acc[...]
        logits = (jnp.dot(pooled, fwref[...], preferred_element_type=jnp.float32)
                  + fbref[...].reshape(1, 10))
        mx = jnp.max(logits, axis=1, keepdims=True)
        lse = jnp.log(jnp.sum(jnp.exp(logits - mx), axis=1, keepdims=True)) + mx
        oref[...] = logits - lse


_k_pool = pl.pallas_call(
    _pool_body,
    grid=(392,),
    in_specs=[
        pl.BlockSpec((128, 128), lambda i: (i, 0)),
        pl.BlockSpec((1, 128), lambda i: (i, 0)),
        pl.BlockSpec((1, 128), lambda i: (i, 0)),
        pl.BlockSpec((128, 10), lambda i: (0, 0)),
        pl.BlockSpec((10,), lambda i: (0,)),
    ],
    out_specs=pl.BlockSpec((1, 10), lambda i: (0, 0)),
    out_shape=jax.ShapeDtypeStruct((1, 10), jnp.float32),
    scratch_shapes=[pltpu.VMEM((1, 128), jnp.float32)],
)


def kernel(x, edge_index, batch, W1, b1, W2, b2, p, fcW, fcb):
    src = edge_index[0]
    dst = edge_index[1]
    x_p = jnp.pad(x, ((0, _NP - _N), (0, 0)))
    degp = _k_deg(dst)
    ht, dinv = _k_mm1(x_p, W1, degp)
    agg1 = _k_agg(ht.reshape(2 * _NP, 32), src, dst)
    g2 = _k_g2(agg1, ht, dinv, b1)
    agg2 = _k_agg(g2.reshape(2 * _NP, 32), src, dst)
    h2, score = _k_h2(agg2, g2, dinv, W2, b2, p)
    w2d, m2d = _k_sel(score.reshape(392, 128))
    return _k_pool(h2, w2d, m2d, fcW, fcb)


# speedup vs baseline: 10.8437x; 10.8437x over previous
import functools

import jax
import jax.numpy as jnp
from jax import lax
from jax.experimental import pallas as pl
from jax.experimental.pallas import tpu as pltpu
from jax.experimental.pallas import tpu_sc as plsc

_N = 50000
_NP = 50176
_E = 800000
_K = 40000
_NS = 16
_NCH = _NP // _NS
_TE = _E // _NS
_NB = _TE // 128
_REM = _TE - _NB * 128
_TED = _E // 2 // _NS
_NBD = _TED // 128
_REMD = _TED - _NBD * 128
_R = 512
_G = _NP // _R
_NEG = -1e30

@functools.cache
def _sc_mesh():
    return plsc.VectorSubcoreMesh(
        core_axis_name="c", subcore_axis_name="s",
        num_cores=2, num_subcores=_NS)


def _zero_rows32(ref, n):
    def body(i, _):
        ref[i, pl.ds(0, 16)] = jnp.zeros((16,), jnp.float32)
        ref[i, pl.ds(16, 16)] = jnp.zeros((16,), jnp.float32)
        return 0
    lax.fori_loop(0, n, body, 0)


def _fill1d(ref, n, val, dtype):
    def body(i, _):
        ref[pl.ds(i * 16, 16)] = jnp.full((16,), val, dtype)
        return 0
    lax.fori_loop(0, n // 16, body, 0)


def _deg_body(dst_hbm, degp, dacc, zbuf, onesv, dstv, onesv2, dstv2):
    c = lax.axis_index("c")
    s = lax.axis_index("s")
    _fill1d(zbuf, 1024, 0.0, jnp.float32)
    _fill1d(onesv, 128, 1.0, jnp.float32)
    _fill1d(onesv2, 48, 1.0, jnp.float32)
    zbase = s * _NCH
    for t in range(_NCH // 1024):
        pltpu.sync_copy(zbuf, dacc.at[pl.ds(zbase + t * 1024, 1024)])
    rem = _NCH % 1024
    if rem:
        pltpu.sync_copy(zbuf.at[pl.ds(0, rem)],
                        dacc.at[pl.ds(zbase + (_NCH // 1024) * 1024, rem)])
    plsc.subcore_barrier()
    tile_base = c * (_E // 2) + s * _TED
    def step(j, _):
        pltpu.sync_copy(dst_hbm.at[pl.ds(tile_base + j * 128, 128)], dstv)
        pltpu.sync_copy(onesv, dacc.at[dstv], add=True)
        return 0
    lax.fori_loop(0, _NBD, step, 0)
    _fill1d(dstv2, 48, _N, jnp.int32)
    pltpu.sync_copy(dst_hbm.at[pl.ds(tile_base + _NBD * 128, _REMD)],
                    dstv2.at[pl.ds(0, _REMD)])
    pltpu.sync_copy(onesv2, dacc.at[dstv2], add=True)
    plsc.subcore_barrier()
    for t in range(_NCH // 1024):
        pltpu.sync_copy(dacc.at[pl.ds(zbase + t * 1024, 1024)], zbuf)
        pltpu.sync_copy(zbuf, degp.at[pl.ds(c * _NP + zbase + t * 1024, 1024)])
    if rem:
        off = (_NCH // 1024) * 1024
        pltpu.sync_copy(dacc.at[pl.ds(zbase + off, rem)], zbuf.at[pl.ds(0, rem)])
        pltpu.sync_copy(zbuf.at[pl.ds(0, rem)],
                        degp.at[pl.ds(c * _NP + zbase + off, rem)])


@functools.cache
def _k_deg():
    return pl.kernel(
        _deg_body,
        out_type=jax.ShapeDtypeStruct((2 * _NP,), jnp.float32),
        mesh=_sc_mesh(),
        compiler_params=pltpu.CompilerParams(use_tc_tiling_on_sc=False),
        scratch_types=[
            pltpu.VMEM_SHARED((_NP,), jnp.float32),
            pltpu.VMEM((1024,), jnp.float32),
            pltpu.VMEM((128,), jnp.float32),
            pltpu.VMEM((128,), jnp.int32),
            pltpu.VMEM((48,), jnp.float32),
            pltpu.VMEM((48,), jnp.int32),
        ],
    )


def _agg_body(hr, srch, dsth, out,
              acc, zrows, srcv, dstv, gidxv, rows,
              srcv2, dstv2, gidxv2, rows2, gsem):
    c = lax.axis_index("c")
    s = lax.axis_index("s")
    _zero_rows32(zrows, 128)
    zbase = s * _NCH
    for t in range(_NCH // 128):
        pltpu.sync_copy(zrows, acc.at[pl.ds(zbase + t * 128, 128)])
    rem = _NCH % 128
    if rem:
        pltpu.sync_copy(zrows.at[pl.ds(0, rem)],
                        acc.at[pl.ds(zbase + (_NCH // 128) * 128, rem)])
    plsc.subcore_barrier()
    tile_base = s * _TE
    def step(j, _):
        base = tile_base + j * 128
        pltpu.sync_copy(srch.at[pl.ds(base, 128)], srcv)
        pltpu.sync_copy(dsth.at[pl.ds(base, 128)], dstv)
        def gi(kk, _2):
            gidxv[pl.ds(kk * 16, 16)] = srcv[pl.ds(kk * 16, 16)] * 2 + c
            return 0
        lax.fori_loop(0, 8, gi, 0)
        pltpu.async_copy(hr.at[gidxv], rows, gsem).wait()
        pltpu.sync_copy(rows, acc.at[dstv], add=True)
        return 0
    lax.fori_loop(0, _NB, step, 0)
    base = tile_base + _NB * 128
    pltpu.sync_copy(srch.at[pl.ds(base, _REM)], srcv2)
    pltpu.sync_copy(dsth.at[pl.ds(base, _REM)], dstv2)
    def gi2(kk, _2):
        gidxv2[pl.ds(kk * 16, 16)] = srcv2[pl.ds(kk * 16, 16)] * 2 + c
        return 0
    lax.fori_loop(0, _REM // 16, gi2, 0)
    pltpu.async_copy(hr.at[gidxv2], rows2, gsem).wait()
    pltpu.sync_copy(rows2, acc.at[dstv2], add=True)
    plsc.subcore_barrier()
    for t in range(_NCH // 128):
        pltpu.sync_copy(acc.at[pl.ds(zbase + t * 128, 128)], zrows)
        pltpu.sync_copy(zrows, out.at[c, pl.ds(zbase + t * 128, 128)])
    if rem:
        off = (_NCH // 128) * 128
        pltpu.sync_copy(acc.at[pl.ds(zbase + off, rem)], zrows.at[pl.ds(0, rem)])
        pltpu.sync_copy(zrows.at[pl.ds(0, rem)],
                        out.at[c, pl.ds(zbase + off, rem)])


@functools.cache
def _k_agg():
    return pl.kernel(
        _agg_body,
        out_type=jax.ShapeDtypeStruct((2, _NP, 32), jnp.float32),
        mesh=_sc_mesh(),
        compiler_params=pltpu.CompilerParams(use_tc_tiling_on_sc=False),
        scratch_types=[
            pltpu.VMEM_SHARED((_NP, 32), jnp.float32),
            pltpu.VMEM((128, 32), jnp.float32),
            pltpu.VMEM((128,), jnp.int32),
            pltpu.VMEM((128,), jnp.int32),
            pltpu.VMEM((128,), jnp.int32),
            pltpu.VMEM((128, 32), jnp.float32),
            pltpu.VMEM((_REM,), jnp.int32),
            pltpu.VMEM((_REM,), jnp.int32),
            pltpu.VMEM((_REM,), jnp.int32),
            pltpu.VMEM((_REM, 32), jnp.float32),
            pltpu.SemaphoreType.DMA,
        ],
    )


def _mm1_body(xref, wref, dref, htref, dinvref):
    i = pl.program_id(0)
    dt = jnp.transpose(dref[...])
    deg = dt[:, 0:1] + dt[:, 1:2] + 1.0
    rows = lax.broadcasted_iota(jnp.int32, (_R, 1), 0) + i * _R
    dinv = jnp.where(rows < _N, lax.rsqrt(deg), 0.0)
    dinvref[...] = dinv
    h = jnp.dot(xref[...], wref[...], preferred_element_type=jnp.float32)
    htref[...] = h * dinv


_k_mm1 = pl.pallas_call(
    _mm1_body,
    grid=(_G,),
    in_specs=[
        pl.BlockSpec((_R, 128), lambda i: (i, 0)),
        pl.BlockSpec((128, 64), lambda i: (0, 0)),
        pl.BlockSpec((2, _R), lambda i: (0, i)),
    ],
    out_specs=[
        pl.BlockSpec((_R, 64), lambda i: (i, 0)),
        pl.BlockSpec((_R, 1), lambda i: (i, 0)),
    ],
    out_shape=[
        jax.ShapeDtypeStruct((_NP, 64), jnp.float32),
        jax.ShapeDtypeStruct((_NP, 1), jnp.float32),
    ],
)


def _g2_body(aref, htref, dinvref, bref, gref):
    i = pl.program_id(0)
    ab = aref[...]
    agg = jnp.concatenate([ab[0], ab[1]], axis=1)
    rows = lax.broadcasted_iota(jnp.int32, (_R, 1), 0) + i * _R
    agg = jnp.where(rows < _N, agg, 0.0)
    dinv = dinvref[...]
    h1 = jnp.maximum(dinv * (agg + htref[...]) + bref[...].reshape(1, 64), 0.0)
    gref[...] = h1 * dinv


_k_g2 = pl.pallas_call(
    _g2_body,
    grid=(_G,),
    in_specs=[
        pl.BlockSpec((2, _R, 32), lambda i: (0, i, 0)),
        pl.BlockSpec((_R, 64), lambda i: (i, 0)),
        pl.BlockSpec((_R, 1), lambda i: (i, 0)),
        pl.BlockSpec((64,), lambda i: (0,)),
    ],
    out_specs=pl.BlockSpec((_R, 64), lambda i: (i, 0)),
    out_shape=jax.ShapeDtypeStruct((_NP, 64), jnp.float32),
)


def _h2_body(aref, gref, dinvref, wref, bref, pref, h2ref, scref):
    i = pl.program_id(0)
    ab = aref[...]
    agg = jnp.concatenate([ab[0], ab[1]], axis=1)
    rows = lax.broadcasted_iota(jnp.int32, (_R, 1), 0) + i * _R
    agg = jnp.where(rows < _N, agg, 0.0)
    dinv = dinvref[...]
    pre = dinv * (agg + gref[...])
    h2 = jnp.dot(pre, wref[...], preferred_element_type=jnp.float32)
    h2 = jnp.maximum(h2 + bref[...].reshape(1, 128), 0.0)
    h2 = jnp.where(rows < _N, h2, 0.0)
    h2ref[...] = h2
    pvec = pref[...].reshape(1, 128)
    pnorm = jnp.sqrt(jnp.sum(pvec * pvec))
    pcol = jnp.transpose(pvec)
    sc = jnp.dot(h2, pcol, preferred_element_type=jnp.float32) / pnorm
    scref[...] = jnp.where(rows < _N, sc, _NEG)


_k_h2 = pl.pallas_call(
    _h2_body,
    grid=(_G,),
    in_specs=[
        pl.BlockSpec((2, _R, 32), lambda i: (0, i, 0)),
        pl.BlockSpec((_R, 64), lambda i: (i, 0)),
        pl.BlockSpec((_R, 1), lambda i: (i, 0)),
        pl.BlockSpec((64, 128), lambda i: (0, 0)),
        pl.BlockSpec((128,), lambda i: (0,)),
        pl.BlockSpec((128,), lambda i: (0,)),
    ],
    out_specs=[
        pl.BlockSpec((_R, 128), lambda i: (i, 0)),
        pl.BlockSpec((_R, 1), lambda i: (i, 0)),
    ],
    out_shape=[
        jax.ShapeDtypeStruct((_NP, 128), jnp.float32),
        jax.ShapeDtypeStruct((_NP, 1), jnp.float32),
    ],
)


def _sel_body(sref, wref, mref):
    s = sref[...]
    ub = lax.bitcast_convert_type(s, jnp.uint32)
    key = jnp.where((ub >> 31) != 0, ~ub, ub | jnp.uint32(0x80000000))
    t = jnp.uint32(0)
    for b in range(31, -1, -1):
        cand = t | (jnp.uint32(1) << b)
        cnt = jnp.sum((key >= cand).astype(jnp.int32))
        t = jnp.where(cnt >= _K, cand, t)
    cgt = jnp.sum((key > t).astype(jnp.int32))
    need = _K - cgt
    eq = key == t
    idx = (lax.broadcasted_iota(jnp.int32, (392, 128), 0) * 128
           + lax.broadcasted_iota(jnp.int32, (392, 128), 1))
    J = jnp.int32(0)
    for b in range(16, -1, -1):
        cand = J + jnp.int32(1 << b)
        cnt = jnp.sum((eq & (idx < cand)).astype(jnp.int32))
        J = jnp.where(cnt <= need, cand, J)
    sel = (key > t) | (eq & (idx < J))
    wref[...] = jnp.tanh(s)
    mref[...] = jnp.where(sel, 0.0, _NEG)


_k_sel = pl.pallas_call(
    _sel_body,
    grid=(1,),
    in_specs=[pl.BlockSpec((392, 128), lambda i: (0, 0))],
    out_specs=[
        pl.BlockSpec((392, 128), lambda i: (0, 0)),
        pl.BlockSpec((392, 128), lambda i: (0, 0)),
    ],
    out_shape=[
        jax.ShapeDtypeStruct((392, 128), jnp.float32),
        jax.ShapeDtypeStruct((392, 128), jnp.float32),
    ],
)


def _pool_body(h2ref, wref, mref, fwref, fbref, oref, acc):
    i = pl.program_id(0)

    @pl.when(i == 0)
    def _init():
        acc[...] = jnp.full((1, 128), _NEG, jnp.float32)

    wcol = jnp.transpose(wref[0])
    mcol = jnp.transpose(mref[0])
    contrib = h2ref[...] * wcol + mcol
    acc[...] = jnp.maximum(acc[...], jnp.max(contrib, axis=0, keepdims=True))

    @pl.when(i == 392 - 1)
    def _fin():
        pooled = acc[...]
        logits = (jnp.dot(pooled, fwref[...], preferred_element_type=jnp.float32)
                  + fbref[...].reshape(1, 10))
        mx = jnp.max(logits, axis=1, keepdims=True)
        lse = jnp.log(jnp.sum(jnp.exp(logits - mx), axis=1, keepdims=True)) + mx
        oref[...] = logits - lse


_k_pool = pl.pallas_call(
    _pool_body,
    grid=(392,),
    in_specs=[
        pl.BlockSpec((128, 128), lambda i: (i, 0)),
        pl.BlockSpec((1, 1, 128), lambda i: (i, 0, 0)),
        pl.BlockSpec((1, 1, 128), lambda i: (i, 0, 0)),
        pl.BlockSpec((128, 10), lambda i: (0, 0)),
        pl.BlockSpec((10,), lambda i: (0,)),
    ],
    out_specs=pl.BlockSpec((1, 10), lambda i: (0, 0)),
    out_shape=jax.ShapeDtypeStruct((1, 10), jnp.float32),
    scratch_shapes=[pltpu.VMEM((1, 128), jnp.float32)],
)


def kernel(x, edge_index, batch, W1, b1, W2, b2, p, fcW, fcb):
    src = edge_index[0]
    dst = edge_index[1]
    x_p = jnp.pad(x, ((0, _NP - _N), (0, 0)))
    degp = _k_deg()(dst).reshape(2, _NP)
    ht, dinv = _k_mm1(x_p, W1, degp)
    agg1 = _k_agg()(ht.reshape(2 * _NP, 32), src, dst)
    g2 = _k_g2(agg1, ht, dinv, b1)
    agg2 = _k_agg()(g2.reshape(2 * _NP, 32), src, dst)
    h2, score = _k_h2(agg2, g2, dinv, W2, b2, p)
    w2d, m2d = _k_sel(score.reshape(392, 128))
    return _k_pool(h2, w2d.reshape(392, 1, 128), m2d.reshape(392, 1, 128),
                   fcW, fcb)
